# baseline (speedup 1.0000x reference)
import functools

import jax
import jax.numpy as jnp
from jax import lax
from jax.experimental import pallas as pl
from jax.experimental.pallas import tpu as pltpu
from jax.experimental.pallas import tpu_sc as plsc

B = 4096
V = 1000000
ET = 2
EMB = 64
EEMB = 16
ATT = 32
NS = 20

_L = 16
_NC = 2
_NSUB = 16
_NW = _NC * _NSUB
_BPW = B // _NW
_PAIRS = _BPW * ET
_NIDX = _PAIRS * NS
_CW = 80
_NCH = _NIDX // _CW
_GRP = 4
_ROWS = V * ET * EEMB // 128
_AGG_R = _PAIRS * EEMB // 128
_NE_R = _BPW * EMB // 128


def _sc_body(nbr_hbm, tgt_hbm, nte_hbm, base_t_hbm, agg_out, ne_out,
             nbr_v, row_v, off_v, ring, agg_v, tgt_v, ne_v,
             sem_g, sem_b):
    wid = lax.axis_index("s") * _NC + lax.axis_index("c")
    pltpu.sync_copy(nbr_hbm.at[pl.ds(wid * _NIDX, _NIDX)], nbr_v)
    pltpu.sync_copy(tgt_hbm, tgt_v)
    tbase = wid * _BPW

    def _bld(c, _):
        lane = lax.iota(jnp.int32, _L)
        step4 = (lane + 12) >> 4
        step8 = (lane + 8) >> 4
        step12 = (lane + 4) >> 4
        et_pat = [None, step4, 1 - step8, step12, 1]
        for p in range(5):
            k = c * 5 + p
            v = nbr_v[pl.ds(k * _L, _L)]
            pat = et_pat[p]
            f = v * ET if pat is None else v * ET + pat
            row_v[c, pl.ds(p * _L, _L)] = f >> 3
            off_v[pl.ds(k * _L, _L)] = (f & 7) << 4
        return 0
    lax.fori_loop(0, _NCH, _bld, 0)


    def _tcol(j):
        tsp = plsc.load_gather(
            tgt_v, [jnp.full((_L,), tbase + j, dtype=jnp.int32)])
        t = jnp.max(tsp)
        return t >> 7, t & 127

    def _sem(slot):
        return sem_g if slot < _GRP else sem_b

    def _bstart(j, slot):
        vt, _ = _tcol(j)
        for cg in range(EMB // 8):
            pltpu.async_copy(
                base_t_hbm.at[pl.ds(cg * 8, 8), pl.ds(vt * 128, 128)],
                ring.at[slot, pl.ds(cg * 8, 8)], _sem(slot))

    def _bproc(j, slot):
        _, vmod = _tcol(j)
        lane = lax.iota(jnp.int32, _L)
        cols = jnp.full((_L,), vmod, dtype=jnp.int32)
        for cg in range(EMB // _L):
            vals = plsc.load_gather(ring.at[slot], [lane + cg * _L, cols])
            ne_v[j >> 1, pl.ds((j & 1) * EMB + cg * _L, _L)] = vals

    def _bwait(slot):
        for cg in range(EMB // 8):
            pltpu.make_async_copy(base_t_hbm.at[pl.ds(cg * 8, 8), pl.ds(0, 128)],
                                  ring.at[slot, pl.ds(cg * 8, 8)],
                                  _sem(slot)).wait()

    def _bgroup(j0, slots, start_next):
        for i, s in enumerate(slots):
            _bwait(s)
        for i, s in enumerate(slots):
            _bproc(j0 + i, s)
        if start_next:
            for i, s in enumerate(slots):
                _bstart(j0 + 2 * _GRP + i, s)

    for i in range(2 * _GRP):
        _bstart(i, i)

    def _bmain(g2, _):
        j0 = g2 * 2 * _GRP
        _bgroup(j0, range(_GRP), True)
        _bgroup(j0 + _GRP, range(_GRP, 2 * _GRP), True)
        return 0
    lax.fori_loop(0, _BPW // (2 * _GRP) - 1, _bmain, 0)
    _bgroup(_BPW - 2 * _GRP, range(_GRP), False)
    _bgroup(_BPW - _GRP, range(_GRP, 2 * _GRP), False)

    pltpu.sync_copy(ne_v, ne_out.at[pl.ds(wid * _NE_R, _NE_R)])

    def _gstart(c, slot):
        pltpu.async_copy(nte_hbm.at[row_v.at[c]], ring.at[slot], _sem(slot))

    def _gwait(slot):
        pltpu.make_async_copy(nte_hbm.at[row_v.at[0]],
                              ring.at[slot], _sem(slot)).wait()

    def _gproc(c, slot):
        lane = lax.iota(jnp.int32, _L)
        for pp in range(_CW // NS):
            p = c * (_CW // NS) + pp
            acc = None
            for s in range(NS):
                r = p * NS + s
                r_loc = pp * NS + s
                offv = plsc.load_gather(
                    off_v, [jnp.full((_L,), r, dtype=jnp.int32)])
                vals = plsc.load_gather(
                    ring.at[slot],
                    [jnp.full((_L,), r_loc, dtype=jnp.int32), offv + lane])
                acc = vals if acc is None else acc + vals
            agg_v[p >> 3, pl.ds((p & 7) * EEMB, EEMB)] = acc

    def _ggroup(c0, slots, start_next):
        for s in slots:
            _gwait(s)
        for i, s in enumerate(slots):
            _gproc(c0 + i, s)
        if start_next:
            for i, s in enumerate(slots):
                _gstart(c0 + 2 * _GRP + i, s)

    for i in range(2 * _GRP):
        _gstart(i, i)

    def _gmain(g2, _):
        c0 = g2 * 2 * _GRP
        _ggroup(c0, range(_GRP), True)
        _ggroup(c0 + _GRP, range(_GRP, 2 * _GRP), True)
        return 0
    lax.fori_loop(0, _NCH // (2 * _GRP) - 1, _gmain, 0)
    _ggroup(_NCH - 2 * _GRP, range(_GRP), False)
    _ggroup(_NCH - _GRP, range(_GRP, 2 * _GRP), False)

    pltpu.sync_copy(agg_v, agg_out.at[pl.ds(wid * _AGG_R, _AGG_R)])


@functools.cache
def _make_sc_gather():
    return functools.partial(
        pl.kernel,
        out_type=[jax.ShapeDtypeStruct((B * ET * EEMB // 128, 128), jnp.float32),
                  jax.ShapeDtypeStruct((B * EMB // 128, 128), jnp.float32)],
        mesh=plsc.VectorSubcoreMesh(core_axis_name="c", subcore_axis_name="s"),
        compiler_params=pltpu.CompilerParams(use_tc_tiling_on_sc=True,
                                             needs_layout_passes=False),
        scratch_types=[
            pltpu.VMEM((_NIDX,), jnp.int32),
            pltpu.VMEM((_NCH, _CW), jnp.int32),
            pltpu.VMEM((_NIDX,), jnp.int32),
            pltpu.VMEM((2 * _GRP, _CW, 128), jnp.float32),
            pltpu.VMEM((_AGG_R, 128), jnp.float32),
            pltpu.VMEM((B,), jnp.int32),
            pltpu.VMEM((_NE_R, 128), jnp.float32),
            pltpu.SemaphoreType.DMA,
            pltpu.SemaphoreType.DMA,
        ],
    )(_sc_body)


def _tc_body(agg_ref, ne_ref, t_ref, s10_ref, s11_ref, s20_ref, s21_ref,
             w0_ref, w1_ref, o_ref):
    a = agg_ref[...] * (1.0 / NS)
    a0 = a[:, :EEMB]
    a1 = a[:, EEMB:]
    t = t_ref[...]
    tn = 1.0 - t

    def _logit(ai):
        h0 = jnp.tanh(jnp.dot(ai, s10_ref[...],
                              preferred_element_type=jnp.float32,
                              precision=lax.Precision.HIGHEST))
        h1 = jnp.tanh(jnp.dot(ai, s11_ref[...],
                              preferred_element_type=jnp.float32,
                              precision=lax.Precision.HIGHEST))
        l0 = jnp.sum(h0 * s20_ref[...], axis=1, keepdims=True)
        l1 = jnp.sum(h1 * s21_ref[...], axis=1, keepdims=True)
        return l0 * tn + l1 * t

    la = _logit(a0)
    lb = _logit(a1)
    m = jnp.maximum(la, lb)
    ea = jnp.exp(la - m)
    eb = jnp.exp(lb - m)
    inv = 1.0 / (ea + eb)
    na = (ea * inv) * a0 + (eb * inv) * a1
    o0 = jnp.dot(na, w0_ref[...], preferred_element_type=jnp.float32,
                 precision=lax.Precision.HIGHEST)
    o1 = jnp.dot(na, w1_ref[...], preferred_element_type=jnp.float32,
                 precision=lax.Precision.HIGHEST)
    allv = ne_ref[...] + o0 * tn + o1 * t
    sq = jnp.sum(allv * allv, axis=1, keepdims=True)
    o_ref[...] = allv * lax.rsqrt(jnp.maximum(sq, 1e-12))


_TC_BLK = 512
_TC_GRID = B // _TC_BLK


def _tc_combine(agg2, ne, tf, s10, s11, s20, s21, w0, w1):
    fixed = lambda i: (0, 0)
    row = lambda i: (i, 0)
    return pl.pallas_call(
        _tc_body,
        grid=(_TC_GRID,),
        in_specs=[
            pl.BlockSpec((_TC_BLK, ET * EEMB), row),
            pl.BlockSpec((_TC_BLK, EMB), row),
            pl.BlockSpec((_TC_BLK, 1), row),
            pl.BlockSpec((EEMB, ATT), fixed),
            pl.BlockSpec((EEMB, ATT), fixed),
            pl.BlockSpec((1, ATT), fixed),
            pl.BlockSpec((1, ATT), fixed),
            pl.BlockSpec((EEMB, EMB), fixed),
            pl.BlockSpec((EEMB, EMB), fixed),
        ],
        out_specs=pl.BlockSpec((_TC_BLK, EMB), row),
        out_shape=jax.ShapeDtypeStruct((B, EMB), jnp.float32),
    )(agg2, ne, tf, s10, s11, s20, s21, w0, w1)


def kernel(targets, types, neighbors, base_node_embeddings,
           node_type_embeddings, trans_weights, trans_weights_s1,
           trans_weights_s2):
    nbr = neighbors.reshape(-1).astype(jnp.int32)
    tgt = targets.astype(jnp.int32)
    nte_r = node_type_embeddings.reshape(_ROWS, 128)
    base_t = base_node_embeddings.T

    agg, ne = _make_sc_gather()(nbr, tgt, nte_r, base_t)

    agg2 = agg.reshape(B, ET * EEMB)
    ne2 = ne.reshape(B, EMB)
    tf = types.astype(jnp.float32).reshape(B, 1)
    return _tc_combine(
        agg2, ne2, tf,
        trans_weights_s1[0], trans_weights_s1[1],
        trans_weights_s2[0].reshape(1, ATT), trans_weights_s2[1].reshape(1, ATT),
        trans_weights[0], trans_weights[1])


# speedup vs baseline: 3.2243x; 3.2243x over previous
import functools

import jax
import jax.numpy as jnp
from jax import lax
from jax.experimental import pallas as pl
from jax.experimental.pallas import tpu as pltpu
from jax.experimental.pallas import tpu_sc as plsc

B = 4096
V = 1000000
ET = 2
EMB = 64
EEMB = 16
ATT = 32
NS = 20

_L = 16
_NC = 2
_NSUB = 16
_NW = _NC * _NSUB
_BPW = B // _NW
_PAIRS = _BPW * ET
_NIDX = _PAIRS * NS
_CHUNK = 128
_NCHUNK = _NIDX // _CHUNK
_GRP = 4
_NE_R = _BPW * EMB // 128


def _sc_base_body(tgt_hbm, base_t_hbm, ne_out, tgt_v, ring, ne_v,
                  sem_a, sem_b):
    wid = lax.axis_index("s") * _NC + lax.axis_index("c")
    pltpu.sync_copy(tgt_hbm, tgt_v)
    tbase = wid * _BPW

    def _tcol(j):
        tsp = plsc.load_gather(
            tgt_v, [jnp.full((_L,), tbase + j, dtype=jnp.int32)])
        t = jnp.max(tsp)
        return t >> 7, t & 127

    def _sem(slot):
        return sem_a if slot < _GRP else sem_b

    def _bstart(j, slot):
        vt, _ = _tcol(j)
        pltpu.async_copy(base_t_hbm.at[:, pl.ds(vt * 128, 128)],
                         ring.at[slot], _sem(slot))

    def _bwait(slot):
        pltpu.make_async_copy(base_t_hbm.at[:, pl.ds(0, 128)],
                              ring.at[slot], _sem(slot)).wait()

    def _bproc(j, slot):
        _, vmod = _tcol(j)
        lane = lax.iota(jnp.int32, _L)
        cols = jnp.full((_L,), vmod, dtype=jnp.int32)
        for cg in range(EMB // _L):
            vals = plsc.load_gather(ring.at[slot], [lane + cg * _L, cols])
            ne_v[j >> 1, pl.ds((j & 1) * EMB + cg * _L, _L)] = vals

    def _bgroup(j0, slots, start_next):
        for s in slots:
            _bwait(s)
        for i, s in enumerate(slots):
            _bproc(j0 + i, s)
        if start_next:
            for i, s in enumerate(slots):
                _bstart(j0 + 2 * _GRP + i, s)

    for i in range(2 * _GRP):
        _bstart(i, i)

    def _bmain(g2, _):
        j0 = g2 * 2 * _GRP
        _bgroup(j0, range(_GRP), True)
        _bgroup(j0 + _GRP, range(_GRP, 2 * _GRP), True)
        return 0
    lax.fori_loop(0, _BPW // (2 * _GRP) - 1, _bmain, 0)
    _bgroup(_BPW - 2 * _GRP, range(_GRP), False)
    _bgroup(_BPW - _GRP, range(_GRP, 2 * _GRP), False)

    pltpu.sync_copy(ne_v, ne_out.at[pl.ds(wid * _NE_R, _NE_R)])


@functools.cache
def _make_sc_base():
    return functools.partial(
        pl.kernel,
        out_type=[jax.ShapeDtypeStruct((B * EMB // 128, 128), jnp.float32)],
        mesh=plsc.VectorSubcoreMesh(core_axis_name="c", subcore_axis_name="s"),
        compiler_params=pltpu.CompilerParams(use_tc_tiling_on_sc=True,
                                             needs_layout_passes=False),
        scratch_types=[
            pltpu.VMEM((B,), jnp.int32),
            pltpu.VMEM((2 * _GRP, EMB, 128), jnp.float32),
            pltpu.VMEM((_NE_R, 128), jnp.float32),
            pltpu.SemaphoreType.DMA,
            pltpu.SemaphoreType.DMA,
        ],
    )(_sc_base_body)


def _sc_agg_body(nbr_hbm, table_hbm, agg_out, nbr_v, idx_v, rows_v, agg_v,
                 sem_g):
    wid = lax.axis_index("s") * _NC + lax.axis_index("c")
    pltpu.sync_copy(nbr_hbm.at[pl.ds(wid * _NIDX, _NIDX)], nbr_v)

    def _sr(sr, _):
        lane = lax.iota(jnp.int32, _L)
        step4 = (lane + 12) >> 4
        step8 = (lane + 8) >> 4
        step12 = (lane + 4) >> 4
        et_pat = [None, step4, 1 - step8, step12, 1]
        for v in range(40):
            src = nbr_v[pl.ds(sr * (40 * _L) + v * _L, _L)]
            pat = et_pat[v % 5]
            idx = src * ET if pat is None else src * ET + pat
            idx_v[sr * 5 + (v // 8), pl.ds((v % 8) * _L, _L)] = idx
        return 0
    lax.fori_loop(0, 8, _sr, 0)

    copies = [pltpu.async_copy(table_hbm.at[idx_v.at[r]],
                               rows_v.at[pl.ds(r * _CHUNK, _CHUNK)], sem_g)
              for r in range(_NCHUNK)]
    for cp in copies:
        cp.wait()

    def _red(j, _):
        acc = rows_v[j * NS]
        for s in range(1, NS):
            acc = acc + rows_v[j * NS + s]
        agg_v[j] = acc
        return 0
    lax.fori_loop(0, _PAIRS, _red, 0)

    pltpu.sync_copy(agg_v, agg_out.at[pl.ds(wid * _PAIRS, _PAIRS)])


@functools.cache
def _make_sc_agg():
    return functools.partial(
        pl.kernel,
        out_type=[jax.ShapeDtypeStruct((B * ET, EEMB), jnp.float32)],
        mesh=plsc.VectorSubcoreMesh(core_axis_name="c", subcore_axis_name="s"),
        compiler_params=pltpu.CompilerParams(use_tc_tiling_on_sc=False),
        scratch_types=[
            pltpu.VMEM((_NIDX,), jnp.int32),
            pltpu.VMEM((_NCHUNK, _CHUNK), jnp.int32),
            pltpu.VMEM((_NIDX, EEMB), jnp.float32),
            pltpu.VMEM((_PAIRS, EEMB), jnp.float32),
            pltpu.SemaphoreType.DMA,
        ],
    )(_sc_agg_body)


def _tc_body(agg_ref, ne_ref, t_ref, s10_ref, s11_ref, s20_ref, s21_ref,
             w0_ref, w1_ref, o_ref):
    a = agg_ref[...] * (1.0 / NS)
    a0 = a[:, :EEMB]
    a1 = a[:, EEMB:]
    t = t_ref[...]
    tn = 1.0 - t

    def _logit(ai):
        h0 = jnp.tanh(jnp.dot(ai, s10_ref[...],
                              preferred_element_type=jnp.float32,
                              precision=lax.Precision.HIGHEST))
        h1 = jnp.tanh(jnp.dot(ai, s11_ref[...],
                              preferred_element_type=jnp.float32,
                              precision=lax.Precision.HIGHEST))
        l0 = jnp.sum(h0 * s20_ref[...], axis=1, keepdims=True)
        l1 = jnp.sum(h1 * s21_ref[...], axis=1, keepdims=True)
        return l0 * tn + l1 * t

    la = _logit(a0)
    lb = _logit(a1)
    m = jnp.maximum(la, lb)
    ea = jnp.exp(la - m)
    eb = jnp.exp(lb - m)
    inv = 1.0 / (ea + eb)
    na = (ea * inv) * a0 + (eb * inv) * a1
    o0 = jnp.dot(na, w0_ref[...], preferred_element_type=jnp.float32,
                 precision=lax.Precision.HIGHEST)
    o1 = jnp.dot(na, w1_ref[...], preferred_element_type=jnp.float32,
                 precision=lax.Precision.HIGHEST)
    allv = ne_ref[...] + o0 * tn + o1 * t
    sq = jnp.sum(allv * allv, axis=1, keepdims=True)
    o_ref[...] = allv * lax.rsqrt(jnp.maximum(sq, 1e-12))


_TC_BLK = 512
_TC_GRID = B // _TC_BLK


def _tc_combine(agg2, ne, tf, s10, s11, s20, s21, w0, w1):
    fixed = lambda i: (0, 0)
    row = lambda i: (i, 0)
    return pl.pallas_call(
        _tc_body,
        grid=(_TC_GRID,),
        in_specs=[
            pl.BlockSpec((_TC_BLK, ET * EEMB), row),
            pl.BlockSpec((_TC_BLK, EMB), row),
            pl.BlockSpec((_TC_BLK, 1), row),
            pl.BlockSpec((EEMB, ATT), fixed),
            pl.BlockSpec((EEMB, ATT), fixed),
            pl.BlockSpec((1, ATT), fixed),
            pl.BlockSpec((1, ATT), fixed),
            pl.BlockSpec((EEMB, EMB), fixed),
            pl.BlockSpec((EEMB, EMB), fixed),
        ],
        out_specs=pl.BlockSpec((_TC_BLK, EMB), row),
        out_shape=jax.ShapeDtypeStruct((B, EMB), jnp.float32),
    )(agg2, ne, tf, s10, s11, s20, s21, w0, w1)


def kernel(targets, types, neighbors, base_node_embeddings,
           node_type_embeddings, trans_weights, trans_weights_s1,
           trans_weights_s2):
    nbr = neighbors.reshape(-1).astype(jnp.int32)
    tgt = targets.astype(jnp.int32)
    table = node_type_embeddings.reshape(V * ET, EEMB)
    base_t = base_node_embeddings.T

    (ne,) = _make_sc_base()(tgt, base_t)
    (agg,) = _make_sc_agg()(nbr, table)

    agg2 = agg.reshape(B, ET * EEMB)
    ne2 = ne.reshape(B, EMB)
    tf = types.astype(jnp.float32).reshape(B, 1)
    return _tc_combine(
        agg2, ne2, tf,
        trans_weights_s1[0], trans_weights_s1[1],
        trans_weights_s2[0].reshape(1, ATT), trans_weights_s2[1].reshape(1, ATT),
        trans_weights[0], trans_weights[1])
